# baseline (speedup 1.0000x reference)
import functools

import jax
import jax.numpy as jnp
from jax import lax
from jax.experimental import pallas as pl
from jax.experimental.pallas import tpu as pltpu
from jax.experimental.pallas import tpu_sc as plsc

_K = 2
_LANES = 16
_NW = 32


def _sc_mesh():
    return plsc.VectorSubcoreMesh(core_axis_name="c", subcore_axis_name="s",
                                  num_cores=2, num_subcores=16)


def _stage_a(x, W_enc, b_enc, W_gate, gamma, BT, E, CAP):
    B, D = x.shape

    def body(x_ref, we_ref, wg_ref, be_ref, g_ref,
             encb_ref, y0_ref, s1_ref, s2_ref, w1_ref, w2_ref, off_ref,
             lt_ref):
        i = pl.program_id(0)

        @pl.when(i == 0)
        def _():
            off_ref[...] = jnp.zeros_like(off_ref)
            r_i = lax.broadcasted_iota(jnp.int32, (BT, BT), 0)
            c_i = lax.broadcasted_iota(jnp.int32, (BT, BT), 1)
            lt_ref[...] = (r_i > c_i).astype(jnp.float32)

        xb = x_ref[...]
        enc = lax.dot_general(xb, we_ref[...], (((1,), (1,)), ((), ())),
                              preferred_element_type=jnp.float32)
        enc = enc + be_ref[...]
        enc16 = enc.astype(jnp.bfloat16).reshape(BT, 2, D // 2)
        encb_ref[...] = pltpu.bitcast(enc16, jnp.int32).reshape(BT, D // 2)
        logits = lax.dot_general(enc, wg_ref[...], (((1,), (1,)), ((), ())),
                                 preferred_element_type=jnp.float32)
        col = lax.broadcasted_iota(jnp.int32, (BT, E), 1)
        m1 = jnp.max(logits, axis=1, keepdims=True)
        i1 = jnp.min(jnp.where(logits == m1, col, E), axis=1, keepdims=True)
        neg = jnp.where(col == i1, -jnp.inf, logits)
        m2 = jnp.max(neg, axis=1, keepdims=True)
        i2 = jnp.min(jnp.where(neg == m2, col, E), axis=1, keepdims=True)
        e2 = jnp.exp(m2 - m1)
        denom = 1.0 + e2 + 1e-12
        y0_ref[...] = enc * ((1.0 + e2) / denom)
        oh1 = (col == i1).astype(jnp.float32)
        oh2 = (col == i2).astype(jnp.float32)
        oh = oh1 + oh2
        off = off_ref[...]
        S = lax.dot_general(lt_ref[...], oh, (((1,), (0,)), ((), ())),
                            preferred_element_type=jnp.float32,
                            precision=lax.Precision.HIGHEST) + off
        off_ref[...] = off + jnp.sum(oh, axis=0, keepdims=True)
        pos1 = jnp.sum(oh1 * S, axis=1, keepdims=True).astype(jnp.int32)
        pos2 = jnp.sum(oh2 * S, axis=1, keepdims=True).astype(jnp.int32)
        dump = jnp.int32(E * CAP)
        s1_ref[...] = jnp.where(pos1 < CAP, i1 * CAP + pos1, dump)
        s2_ref[...] = jnp.where(pos2 < CAP, i2 * CAP + pos2, dump)
        g1 = jnp.sum(oh1 * g_ref[...], axis=1, keepdims=True)
        g2 = jnp.sum(oh2 * g_ref[...], axis=1, keepdims=True)
        w1_ref[...] = g1 / denom
        w2_ref[...] = (e2 / denom) * g2

    grid = (B // BT,)
    return pl.pallas_call(
        body,
        grid=grid,
        in_specs=[
            pl.BlockSpec((BT, D), lambda i: (i, 0)),
            pl.BlockSpec((D, D), lambda i: (0, 0)),
            pl.BlockSpec((E, D), lambda i: (0, 0)),
            pl.BlockSpec((1, D), lambda i: (0, 0)),
            pl.BlockSpec((1, E), lambda i: (0, 0)),
        ],
        out_specs=[
            pl.BlockSpec((BT, D // 2), lambda i: (i, 0)),
            pl.BlockSpec((BT, D), lambda i: (i, 0)),
            pl.BlockSpec((BT, 1), lambda i: (i, 0)),
            pl.BlockSpec((BT, 1), lambda i: (i, 0)),
            pl.BlockSpec((BT, 1), lambda i: (i, 0)),
            pl.BlockSpec((BT, 1), lambda i: (i, 0)),
        ],
        out_shape=[
            jax.ShapeDtypeStruct((B, D // 2), jnp.int32),
            jax.ShapeDtypeStruct((B, D), jnp.float32),
            jax.ShapeDtypeStruct((B, 1), jnp.int32),
            jax.ShapeDtypeStruct((B, 1), jnp.int32),
            jax.ShapeDtypeStruct((B, 1), jnp.float32),
            jax.ShapeDtypeStruct((B, 1), jnp.float32),
        ],
        scratch_shapes=[pltpu.VMEM((1, E), jnp.float32),
                        pltpu.VMEM((BT, BT), jnp.float32)],
    )(x, W_enc, W_gate, b_enc.reshape(1, D), gamma.reshape(1, E))


def _dispatch(sidx0, sidx1, wg0, wg1, enc, B, D, E, CAP):
    TPW = B // _NW
    TCH = 128
    NCH = TPW // TCH
    DP = D // 2

    @functools.partial(
        pl.kernel,
        out_type=(jax.ShapeDtypeStruct(((E + 1) * CAP, DP), jnp.int32),
                  jax.ShapeDtypeStruct(((E + 1) * CAP,), jnp.float32)),
        mesh=_sc_mesh(),
        scratch_types=(
            [pltpu.VMEM((TCH,), jnp.int32)] * 4
            + [pltpu.VMEM((TCH,), jnp.float32)] * 4
            + [pltpu.VMEM((TCH, DP), jnp.int32)] * 2
            + [pltpu.SemaphoreType.DMA] * 4
        ),
    )
    def k(s0_hbm, s1_hbm, w0_hbm, w1_hbm, enc_hbm, buf_hbm, wsl_hbm,
          i0a, i1a, i0b, i1b, w0a, w1a, w0b, w1b, rowsa, rowsb,
          g0, g1, s0, s1):
        wid = lax.axis_index("s") * 2 + lax.axis_index("c")
        bufs = ((i0a, i1a, w0a, w1a, rowsa, g0, s0),
                (i0b, i1b, w0b, w1b, rowsb, g1, s1))

        def start(c, bs):
            i0_v, i1_v, w0_v, w1_v, rows_v, g_sem, _ = bs
            t0 = wid * TPW + c * TCH
            pltpu.async_copy(s0_hbm.at[pl.ds(t0, TCH)], i0_v, g_sem)
            pltpu.async_copy(s1_hbm.at[pl.ds(t0, TCH)], i1_v, g_sem)
            pltpu.async_copy(w0_hbm.at[pl.ds(t0, TCH)], w0_v, g_sem)
            pltpu.async_copy(w1_hbm.at[pl.ds(t0, TCH)], w1_v, g_sem)
            pltpu.async_copy(enc_hbm.at[pl.ds(t0, TCH)], rows_v, g_sem)

        start(0, bufs[0])
        start(1, bufs[1])
        for c in range(NCH):
            i0_v, i1_v, w0_v, w1_v, rows_v, g_sem, s_sem = bufs[c]
            pltpu.make_async_copy(s0_hbm.at[pl.ds(0, TCH)], i0_v, g_sem).wait()
            pltpu.make_async_copy(s1_hbm.at[pl.ds(0, TCH)], i1_v, g_sem).wait()
            pltpu.make_async_copy(w0_hbm.at[pl.ds(0, TCH)], w0_v, g_sem).wait()
            pltpu.make_async_copy(w1_hbm.at[pl.ds(0, TCH)], w1_v, g_sem).wait()
            pltpu.make_async_copy(enc_hbm.at[pl.ds(0, TCH)], rows_v,
                                  g_sem).wait()
            pltpu.async_copy(rows_v, buf_hbm.at[i0_v], s_sem)
            pltpu.async_copy(rows_v, buf_hbm.at[i1_v], s_sem)
            pltpu.async_copy(w0_v, wsl_hbm.at[i0_v], s_sem)
            pltpu.async_copy(w1_v, wsl_hbm.at[i1_v], s_sem)
        for c in range(NCH):
            i0_v, i1_v, w0_v, w1_v, rows_v, g_sem, s_sem = bufs[c]
            pltpu.make_async_copy(rows_v, buf_hbm.at[pl.ds(0, TCH)],
                                  s_sem).wait()
            pltpu.make_async_copy(rows_v, buf_hbm.at[pl.ds(0, TCH)],
                                  s_sem).wait()
            pltpu.make_async_copy(w0_v, wsl_hbm.at[pl.ds(0, TCH)],
                                  s_sem).wait()
            pltpu.make_async_copy(w1_v, wsl_hbm.at[pl.ds(0, TCH)],
                                  s_sem).wait()

    return k(sidx0, sidx1, wg0, wg1, enc)


def _experts(buf, U, V, wslot, E, CAP, D, R):
    def body(buf_ref, u_ref, v_ref, w_ref, out_ref):
        e = pl.program_id(0)
        xb = pltpu.bitcast(buf_ref[...].reshape(CAP, 1, D // 2),
                           jnp.bfloat16).reshape(CAP, D)
        z = jnp.dot(xb, u_ref[0], preferred_element_type=jnp.float32)
        h = z * (1.0 / (1.0 + jnp.exp(-z)))
        dlt = jnp.dot(h.astype(jnp.bfloat16), v_ref[0],
                      preferred_element_type=jnp.float32)
        dlt = dlt * w_ref[0]
        out_ref[...] = jnp.where(e == E, 0.0, dlt)

    return pl.pallas_call(
        body,
        grid=(E + 1,),
        in_specs=[
            pl.BlockSpec((CAP, D // 2), lambda e: (e, 0)),
            pl.BlockSpec((1, D, R), lambda e: (jnp.minimum(e, E - 1), 0, 0)),
            pl.BlockSpec((1, R, D), lambda e: (jnp.minimum(e, E - 1), 0, 0)),
            pl.BlockSpec((1, CAP, 1), lambda e: (e, 0, 0)),
        ],
        out_specs=pl.BlockSpec((CAP, D), lambda e: (e, 0)),
        out_shape=jax.ShapeDtypeStruct(((E + 1) * CAP, D), jnp.float32),
    )(buf, U.astype(jnp.bfloat16), V.astype(jnp.bfloat16),
      wslot.reshape(E + 1, CAP, 1))


def _combine(sidx0, sidx1, y0, dscaled, B, D):
    TCH = 16
    TPW = B // _NW
    NCH = TPW // TCH
    NV = D // _LANES

    @functools.partial(
        pl.kernel,
        out_type=jax.ShapeDtypeStruct((B, D), jnp.float32),
        mesh=_sc_mesh(),
        scratch_types=[
            pltpu.VMEM((TCH,), jnp.int32),
            pltpu.VMEM((TCH,), jnp.int32),
            pltpu.VMEM((TCH,), jnp.int32),
            pltpu.VMEM((TCH,), jnp.int32),
            pltpu.VMEM((TCH, D), jnp.float32),
            pltpu.VMEM((TCH, D), jnp.float32),
            pltpu.VMEM((TCH, D), jnp.float32),
            pltpu.VMEM((TCH, D), jnp.float32),
            pltpu.VMEM((TCH, D), jnp.float32),
            pltpu.VMEM((TCH, D), jnp.float32),
            pltpu.VMEM((TCH, D), jnp.float32),
            pltpu.VMEM((TCH, D), jnp.float32),
            pltpu.SemaphoreType.DMA,
            pltpu.SemaphoreType.DMA,
            pltpu.SemaphoreType.DMA,
            pltpu.SemaphoreType.DMA,
        ],
    )
    def k(s0_hbm, s1_hbm, y0_hbm, dsc_hbm, y_hbm,
          i0a, i1a, i0b, i1b, d0a, d1a, y0a, yoa, d0b, d1b, y0b, yob,
          ga, gb, wa, wb):
        wid = lax.axis_index("s") * 2 + lax.axis_index("c")
        t0 = wid * TPW
        bufs = ((i0a, i1a, d0a, d1a, y0a, yoa, ga, wa),
                (i0b, i1b, d0b, d1b, y0b, yob, gb, wb))

        def start(c, bs):
            i0_v, i1_v, d0_v, d1_v, y0_v, _, g_sem, _ = bs
            tb = t0 + c * TCH
            pltpu.sync_copy(s0_hbm.at[pl.ds(tb, TCH)], i0_v)
            pltpu.sync_copy(s1_hbm.at[pl.ds(tb, TCH)], i1_v)
            pltpu.async_copy(dsc_hbm.at[i0_v], d0_v, g_sem)
            pltpu.async_copy(dsc_hbm.at[i1_v], d1_v, g_sem)
            pltpu.async_copy(y0_hbm.at[pl.ds(tb, TCH)], y0_v, g_sem)

        for b in range(2):
            start(b, bufs[b])

        def phase(j, c, bs):
            _, _, d0_v, d1_v, y0_v, yo_v, g_sem, w_sem = bs
            tb = t0 + c * TCH
            pltpu.make_async_copy(dsc_hbm.at[pl.ds(0, TCH)], d0_v, g_sem).wait()
            pltpu.make_async_copy(dsc_hbm.at[pl.ds(0, TCH)], d1_v, g_sem).wait()
            pltpu.make_async_copy(y0_hbm.at[pl.ds(0, TCH)], y0_v, g_sem).wait()

            @pl.when(j > 0)
            def _():
                pltpu.make_async_copy(yo_v, y_hbm.at[pl.ds(0, TCH)],
                                      w_sem).wait()

            def vloop(q, _):
                t = q // NV
                o = (q % NV) * _LANES
                yo_v[t, pl.ds(o, _LANES)] = (y0_v[t, pl.ds(o, _LANES)]
                                             + d0_v[t, pl.ds(o, _LANES)]
                                             + d1_v[t, pl.ds(o, _LANES)])
                return 0

            lax.fori_loop(0, TCH * NV, vloop, 0)
            pltpu.async_copy(yo_v, y_hbm.at[pl.ds(tb, TCH)], w_sem)

            @pl.when(c + 2 < NCH)
            def _():
                start(c + 2, bs)

        def body(j, _):
            phase(j, 2 * j, bufs[0])
            phase(j, 2 * j + 1, bufs[1])
            return 0

        lax.fori_loop(0, NCH // 2, body, 0)
        for b in range(2):
            _, _, _, _, _, yo_v, _, w_sem = bufs[b]
            pltpu.make_async_copy(yo_v, y_hbm.at[pl.ds(0, TCH)], w_sem).wait()

    return k(sidx0, sidx1, y0, dscaled)


def kernel(x, W_enc, b_enc, W_gate, U, V, gamma):
    B, D = x.shape
    E = W_gate.shape[0]
    R = U.shape[2]
    CAP = int(1.25 * B * _K / E)
    BT = 512

    enc_bf, y0, s1, s2, w1, w2 = _stage_a(x, W_enc, b_enc, W_gate, gamma,
                                          BT, E, CAP)
    sidx0 = s1.reshape(B)
    sidx1 = s2.reshape(B)
    wg0 = w1.reshape(B)
    wg1 = w2.reshape(B)
    buf, wslot = _dispatch(sidx0, sidx1, wg0, wg1, enc_bf, B, D, E, CAP)
    dscaled = _experts(buf, U, V, wslot, E, CAP, D, R)
    y = _combine(sidx0, sidx1, y0, dscaled, B, D)
    return y


# speedup vs baseline: 1.0896x; 1.0896x over previous
import functools

import jax
import jax.numpy as jnp
from jax import lax
from jax.experimental import pallas as pl
from jax.experimental.pallas import tpu as pltpu
from jax.experimental.pallas import tpu_sc as plsc

_K = 2
_LANES = 16
_NW = 32


def _sc_mesh():
    return plsc.VectorSubcoreMesh(core_axis_name="c", subcore_axis_name="s",
                                  num_cores=2, num_subcores=16)


def _stage_a(x, W_enc, b_enc, W_gate, gamma, BT, E, CAP):
    B, D = x.shape

    def body(x_ref, we_ref, wg_ref, be_ref, g_ref,
             encb_ref, y0_ref, s1_ref, s2_ref, w1_ref, w2_ref, off_ref,
             lt_ref):
        i = pl.program_id(0)

        @pl.when(i == 0)
        def _():
            off_ref[...] = jnp.zeros_like(off_ref)
            r_i = lax.broadcasted_iota(jnp.int32, (BT, BT), 0)
            c_i = lax.broadcasted_iota(jnp.int32, (BT, BT), 1)
            lt_ref[...] = (r_i > c_i).astype(jnp.float32)

        xb = x_ref[...]
        enc = lax.dot_general(xb, we_ref[...], (((1,), (1,)), ((), ())),
                              preferred_element_type=jnp.float32)
        enc = enc + be_ref[...]
        enc16 = enc.astype(jnp.bfloat16).reshape(BT, 2, D // 2)
        encb_ref[...] = pltpu.bitcast(enc16, jnp.int32).reshape(BT, D // 2)
        logits = lax.dot_general(enc, wg_ref[...], (((1,), (1,)), ((), ())),
                                 preferred_element_type=jnp.float32)
        col = lax.broadcasted_iota(jnp.int32, (BT, E), 1)
        m1 = jnp.max(logits, axis=1, keepdims=True)
        i1 = jnp.min(jnp.where(logits == m1, col, E), axis=1, keepdims=True)
        neg = jnp.where(col == i1, -jnp.inf, logits)
        m2 = jnp.max(neg, axis=1, keepdims=True)
        i2 = jnp.min(jnp.where(neg == m2, col, E), axis=1, keepdims=True)
        e2 = jnp.exp(m2 - m1)
        denom = 1.0 + e2 + 1e-12
        y0_ref[...] = enc * ((1.0 + e2) / denom)
        oh1 = (col == i1).astype(jnp.float32)
        oh2 = (col == i2).astype(jnp.float32)
        oh = oh1 + oh2
        off = off_ref[...]
        S = lax.dot_general(lt_ref[...], oh, (((1,), (0,)), ((), ())),
                            preferred_element_type=jnp.float32,
                            precision=lax.Precision.HIGHEST) + off
        off_ref[...] = off + jnp.sum(oh, axis=0, keepdims=True)
        pos1 = jnp.sum(oh1 * S, axis=1, keepdims=True).astype(jnp.int32)
        pos2 = jnp.sum(oh2 * S, axis=1, keepdims=True).astype(jnp.int32)
        dump = jnp.int32(E * CAP)
        s1_ref[...] = jnp.where(pos1 < CAP, i1 * CAP + pos1, dump)
        s2_ref[...] = jnp.where(pos2 < CAP, i2 * CAP + pos2, dump)
        g1 = jnp.sum(oh1 * g_ref[...], axis=1, keepdims=True)
        g2 = jnp.sum(oh2 * g_ref[...], axis=1, keepdims=True)
        w1_ref[...] = g1 / denom
        w2_ref[...] = (e2 / denom) * g2

    grid = (B // BT,)
    return pl.pallas_call(
        body,
        grid=grid,
        in_specs=[
            pl.BlockSpec((BT, D), lambda i: (i, 0)),
            pl.BlockSpec((D, D), lambda i: (0, 0)),
            pl.BlockSpec((E, D), lambda i: (0, 0)),
            pl.BlockSpec((1, D), lambda i: (0, 0)),
            pl.BlockSpec((1, E), lambda i: (0, 0)),
        ],
        out_specs=[
            pl.BlockSpec((BT, D // 2), lambda i: (i, 0)),
            pl.BlockSpec((BT, D), lambda i: (i, 0)),
            pl.BlockSpec((BT, 1), lambda i: (i, 0)),
            pl.BlockSpec((BT, 1), lambda i: (i, 0)),
            pl.BlockSpec((BT, 1), lambda i: (i, 0)),
            pl.BlockSpec((BT, 1), lambda i: (i, 0)),
        ],
        out_shape=[
            jax.ShapeDtypeStruct((B, D // 2), jnp.int32),
            jax.ShapeDtypeStruct((B, D), jnp.float32),
            jax.ShapeDtypeStruct((B, 1), jnp.int32),
            jax.ShapeDtypeStruct((B, 1), jnp.int32),
            jax.ShapeDtypeStruct((B, 1), jnp.float32),
            jax.ShapeDtypeStruct((B, 1), jnp.float32),
        ],
        scratch_shapes=[pltpu.VMEM((1, E), jnp.float32),
                        pltpu.VMEM((BT, BT), jnp.float32)],
    )(x, W_enc, W_gate, b_enc.reshape(1, D), gamma.reshape(1, E))


def _dispatch(sidx0, sidx1, wg0, wg1, enc, B, D, E, CAP):
    TPW = B // _NW
    TCH = 128
    NCH = TPW // TCH
    DP = D // 2

    @functools.partial(
        pl.kernel,
        out_type=(jax.ShapeDtypeStruct(((E + 4) * CAP, DP), jnp.int32),
                  jax.ShapeDtypeStruct(((E + 4) * CAP,), jnp.float32)),
        mesh=_sc_mesh(),
        scratch_types=(
            [pltpu.VMEM((TCH,), jnp.int32)] * 4
            + [pltpu.VMEM((TCH,), jnp.float32)] * 4
            + [pltpu.VMEM((TCH, DP), jnp.int32)] * 2
            + [pltpu.SemaphoreType.DMA] * 4
        ),
    )
    def k(s0_hbm, s1_hbm, w0_hbm, w1_hbm, enc_hbm, buf_hbm, wsl_hbm,
          i0a, i1a, i0b, i1b, w0a, w1a, w0b, w1b, rowsa, rowsb,
          g0, g1, s0, s1):
        wid = lax.axis_index("s") * 2 + lax.axis_index("c")
        bufs = ((i0a, i1a, w0a, w1a, rowsa, g0, s0),
                (i0b, i1b, w0b, w1b, rowsb, g1, s1))

        def start(c, bs):
            i0_v, i1_v, w0_v, w1_v, rows_v, g_sem, _ = bs
            t0 = wid * TPW + c * TCH
            pltpu.async_copy(s0_hbm.at[pl.ds(t0, TCH)], i0_v, g_sem)
            pltpu.async_copy(s1_hbm.at[pl.ds(t0, TCH)], i1_v, g_sem)
            pltpu.async_copy(w0_hbm.at[pl.ds(t0, TCH)], w0_v, g_sem)
            pltpu.async_copy(w1_hbm.at[pl.ds(t0, TCH)], w1_v, g_sem)
            pltpu.async_copy(enc_hbm.at[pl.ds(t0, TCH)], rows_v, g_sem)

        start(0, bufs[0])
        start(1, bufs[1])
        for c in range(NCH):
            i0_v, i1_v, w0_v, w1_v, rows_v, g_sem, s_sem = bufs[c]
            pltpu.make_async_copy(s0_hbm.at[pl.ds(0, TCH)], i0_v, g_sem).wait()
            pltpu.make_async_copy(s1_hbm.at[pl.ds(0, TCH)], i1_v, g_sem).wait()
            pltpu.make_async_copy(w0_hbm.at[pl.ds(0, TCH)], w0_v, g_sem).wait()
            pltpu.make_async_copy(w1_hbm.at[pl.ds(0, TCH)], w1_v, g_sem).wait()
            pltpu.make_async_copy(enc_hbm.at[pl.ds(0, TCH)], rows_v,
                                  g_sem).wait()
            pltpu.async_copy(rows_v, buf_hbm.at[i0_v], s_sem)
            pltpu.async_copy(rows_v, buf_hbm.at[i1_v], s_sem)
            pltpu.async_copy(w0_v, wsl_hbm.at[i0_v], s_sem)
            pltpu.async_copy(w1_v, wsl_hbm.at[i1_v], s_sem)
        for c in range(NCH):
            i0_v, i1_v, w0_v, w1_v, rows_v, g_sem, s_sem = bufs[c]
            pltpu.make_async_copy(rows_v, buf_hbm.at[pl.ds(0, TCH)],
                                  s_sem).wait()
            pltpu.make_async_copy(rows_v, buf_hbm.at[pl.ds(0, TCH)],
                                  s_sem).wait()
            pltpu.make_async_copy(w0_v, wsl_hbm.at[pl.ds(0, TCH)],
                                  s_sem).wait()
            pltpu.make_async_copy(w1_v, wsl_hbm.at[pl.ds(0, TCH)],
                                  s_sem).wait()

    return k(sidx0, sidx1, wg0, wg1, enc)


def _experts(buf, U, V, wslot, E, CAP, D, R):
    EG = 4
    NG = E // EG

    def body(buf_ref, u_ref, v_ref, w_ref, out_ref):
        e = pl.program_id(0)
        xb = pltpu.bitcast(buf_ref[...].reshape(EG * CAP, 1, D // 2),
                           jnp.bfloat16).reshape(EG * CAP, D)
        outs = []
        for g in range(EG):
            xg = xb[g * CAP:(g + 1) * CAP]
            z = jnp.dot(xg, u_ref[g], preferred_element_type=jnp.float32)
            h = z * (1.0 / (1.0 + jnp.exp(-z)))
            outs.append(jnp.dot(h.astype(jnp.bfloat16), v_ref[g],
                                preferred_element_type=jnp.float32))
        dlt = jnp.concatenate(outs, axis=0) * w_ref[0]
        out_ref[...] = jnp.where(e == NG, 0.0, dlt)

    return pl.pallas_call(
        body,
        grid=(NG + 1,),
        in_specs=[
            pl.BlockSpec((EG * CAP, D // 2), lambda e: (e, 0)),
            pl.BlockSpec((EG, D, R), lambda e: (jnp.minimum(e, NG - 1), 0, 0)),
            pl.BlockSpec((EG, R, D), lambda e: (jnp.minimum(e, NG - 1), 0, 0)),
            pl.BlockSpec((1, EG * CAP, 1), lambda e: (e, 0, 0)),
        ],
        out_specs=pl.BlockSpec((EG * CAP, D), lambda e: (e, 0)),
        out_shape=jax.ShapeDtypeStruct(((E + EG) * CAP, D), jnp.float32),
    )(buf, U.astype(jnp.bfloat16), V.astype(jnp.bfloat16),
      wslot.reshape(E // EG + 1, EG * CAP, 1))


def _combine(sidx0, sidx1, y0, dscaled, B, D):
    TCH = 16
    TPW = B // _NW
    NCH = TPW // TCH
    NV = D // _LANES

    @functools.partial(
        pl.kernel,
        out_type=jax.ShapeDtypeStruct((B, D), jnp.float32),
        mesh=_sc_mesh(),
        scratch_types=[
            pltpu.VMEM((TCH,), jnp.int32),
            pltpu.VMEM((TCH,), jnp.int32),
            pltpu.VMEM((TCH,), jnp.int32),
            pltpu.VMEM((TCH,), jnp.int32),
            pltpu.VMEM((TCH, D), jnp.float32),
            pltpu.VMEM((TCH, D), jnp.float32),
            pltpu.VMEM((TCH, D), jnp.float32),
            pltpu.VMEM((TCH, D), jnp.float32),
            pltpu.VMEM((TCH, D), jnp.float32),
            pltpu.VMEM((TCH, D), jnp.float32),
            pltpu.VMEM((TCH, D), jnp.float32),
            pltpu.VMEM((TCH, D), jnp.float32),
            pltpu.SemaphoreType.DMA,
            pltpu.SemaphoreType.DMA,
            pltpu.SemaphoreType.DMA,
            pltpu.SemaphoreType.DMA,
        ],
    )
    def k(s0_hbm, s1_hbm, y0_hbm, dsc_hbm, y_hbm,
          i0a, i1a, i0b, i1b, d0a, d1a, y0a, yoa, d0b, d1b, y0b, yob,
          ga, gb, wa, wb):
        wid = lax.axis_index("s") * 2 + lax.axis_index("c")
        t0 = wid * TPW
        bufs = ((i0a, i1a, d0a, d1a, y0a, yoa, ga, wa),
                (i0b, i1b, d0b, d1b, y0b, yob, gb, wb))

        def start(c, bs):
            i0_v, i1_v, d0_v, d1_v, y0_v, _, g_sem, _ = bs
            tb = t0 + c * TCH
            pltpu.sync_copy(s0_hbm.at[pl.ds(tb, TCH)], i0_v)
            pltpu.sync_copy(s1_hbm.at[pl.ds(tb, TCH)], i1_v)
            pltpu.async_copy(dsc_hbm.at[i0_v], d0_v, g_sem)
            pltpu.async_copy(dsc_hbm.at[i1_v], d1_v, g_sem)
            pltpu.async_copy(y0_hbm.at[pl.ds(tb, TCH)], y0_v, g_sem)

        for b in range(2):
            start(b, bufs[b])

        def phase(j, c, bs):
            _, _, d0_v, d1_v, y0_v, yo_v, g_sem, w_sem = bs
            tb = t0 + c * TCH
            pltpu.make_async_copy(dsc_hbm.at[pl.ds(0, TCH)], d0_v, g_sem).wait()
            pltpu.make_async_copy(dsc_hbm.at[pl.ds(0, TCH)], d1_v, g_sem).wait()
            pltpu.make_async_copy(y0_hbm.at[pl.ds(0, TCH)], y0_v, g_sem).wait()

            @pl.when(j > 0)
            def _():
                pltpu.make_async_copy(yo_v, y_hbm.at[pl.ds(0, TCH)],
                                      w_sem).wait()

            def vloop(q, _):
                t = q // NV
                o = (q % NV) * _LANES
                yo_v[t, pl.ds(o, _LANES)] = (y0_v[t, pl.ds(o, _LANES)]
                                             + d0_v[t, pl.ds(o, _LANES)]
                                             + d1_v[t, pl.ds(o, _LANES)])
                return 0

            lax.fori_loop(0, TCH * NV, vloop, 0)
            pltpu.async_copy(yo_v, y_hbm.at[pl.ds(tb, TCH)], w_sem)

            @pl.when(c + 2 < NCH)
            def _():
                start(c + 2, bs)

        def body(j, _):
            phase(j, 2 * j, bufs[0])
            phase(j, 2 * j + 1, bufs[1])
            return 0

        lax.fori_loop(0, NCH // 2, body, 0)
        for b in range(2):
            _, _, _, _, _, yo_v, _, w_sem = bufs[b]
            pltpu.make_async_copy(yo_v, y_hbm.at[pl.ds(0, TCH)], w_sem).wait()

    return k(sidx0, sidx1, y0, dscaled)


def kernel(x, W_enc, b_enc, W_gate, U, V, gamma):
    B, D = x.shape
    E = W_gate.shape[0]
    R = U.shape[2]
    CAP = int(1.25 * B * _K / E)
    BT = 512

    enc_bf, y0, s1, s2, w1, w2 = _stage_a(x, W_enc, b_enc, W_gate, gamma,
                                          BT, E, CAP)
    sidx0 = s1.reshape(B)
    sidx1 = s2.reshape(B)
    wg0 = w1.reshape(B)
    wg1 = w2.reshape(B)
    buf, wslot = _dispatch(sidx0, sidx1, wg0, wg1, enc_bf, B, D, E, CAP)
    dscaled = _experts(buf, U, V, wslot, E, CAP, D, R)
    y = _combine(sidx0, sidx1, y0, dscaled, B, D)
    return y


# speedup vs baseline: 1.1942x; 1.0960x over previous
import functools

import jax
import jax.numpy as jnp
from jax import lax
from jax.experimental import pallas as pl
from jax.experimental.pallas import tpu as pltpu
from jax.experimental.pallas import tpu_sc as plsc

_K = 2
_LANES = 16
_NW = 32


def _sc_mesh():
    return plsc.VectorSubcoreMesh(core_axis_name="c", subcore_axis_name="s",
                                  num_cores=2, num_subcores=16)


def _stage_a(x, W_enc, b_enc, W_gate, gamma, BT, E, CAP):
    B, D = x.shape

    def body(x_ref, we_ref, wg_ref, be_ref, g_ref,
             encb_ref, y0_ref, s1_ref, s2_ref, w1_ref, w2_ref, off_ref,
             lt_ref):
        i = pl.program_id(0)

        @pl.when(i == 0)
        def _():
            off_ref[...] = jnp.zeros_like(off_ref)
            r_i = lax.broadcasted_iota(jnp.int32, (BT, BT), 0)
            c_i = lax.broadcasted_iota(jnp.int32, (BT, BT), 1)
            lt_ref[...] = (r_i > c_i).astype(jnp.float32)

        xb = x_ref[...]
        enc = lax.dot_general(xb, we_ref[...], (((1,), (1,)), ((), ())),
                              preferred_element_type=jnp.float32)
        enc = enc + be_ref[...]
        enc16 = enc.astype(jnp.bfloat16).reshape(BT, 2, D // 2)
        encb_ref[...] = pltpu.bitcast(enc16, jnp.int32).reshape(BT, D // 2)
        logits = lax.dot_general(enc, wg_ref[...], (((1,), (1,)), ((), ())),
                                 preferred_element_type=jnp.float32)
        col = lax.broadcasted_iota(jnp.int32, (BT, E), 1)
        m1 = jnp.max(logits, axis=1, keepdims=True)
        i1 = jnp.min(jnp.where(logits == m1, col, E), axis=1, keepdims=True)
        neg = jnp.where(col == i1, -jnp.inf, logits)
        m2 = jnp.max(neg, axis=1, keepdims=True)
        i2 = jnp.min(jnp.where(neg == m2, col, E), axis=1, keepdims=True)
        e2 = jnp.exp(m2 - m1)
        denom = 1.0 + e2 + 1e-12
        y0_ref[...] = enc * ((1.0 + e2) / denom)
        oh1 = (col == i1).astype(jnp.float32)
        oh2 = (col == i2).astype(jnp.float32)
        oh = oh1 + oh2
        off = off_ref[...]
        S = lax.dot_general(lt_ref[...], oh, (((1,), (0,)), ((), ())),
                            preferred_element_type=jnp.float32) + off
        off_ref[...] = off + jnp.sum(oh, axis=0, keepdims=True)
        pos1 = jnp.sum(oh1 * S, axis=1, keepdims=True).astype(jnp.int32)
        pos2 = jnp.sum(oh2 * S, axis=1, keepdims=True).astype(jnp.int32)
        dump = jnp.int32(E * CAP)
        s1_ref[...] = jnp.where(pos1 < CAP, i1 * CAP + pos1, dump)
        s2_ref[...] = jnp.where(pos2 < CAP, i2 * CAP + pos2, dump)
        g1 = jnp.sum(oh1 * g_ref[...], axis=1, keepdims=True)
        g2 = jnp.sum(oh2 * g_ref[...], axis=1, keepdims=True)
        w1_ref[...] = g1 / denom
        w2_ref[...] = (e2 / denom) * g2

    grid = (B // BT,)
    return pl.pallas_call(
        body,
        grid=grid,
        in_specs=[
            pl.BlockSpec((BT, D), lambda i: (i, 0)),
            pl.BlockSpec((D, D), lambda i: (0, 0)),
            pl.BlockSpec((E, D), lambda i: (0, 0)),
            pl.BlockSpec((1, D), lambda i: (0, 0)),
            pl.BlockSpec((1, E), lambda i: (0, 0)),
        ],
        out_specs=[
            pl.BlockSpec((BT, D // 2), lambda i: (i, 0)),
            pl.BlockSpec((BT, D), lambda i: (i, 0)),
            pl.BlockSpec((BT, 1), lambda i: (i, 0)),
            pl.BlockSpec((BT, 1), lambda i: (i, 0)),
            pl.BlockSpec((BT, 1), lambda i: (i, 0)),
            pl.BlockSpec((BT, 1), lambda i: (i, 0)),
        ],
        out_shape=[
            jax.ShapeDtypeStruct((B, D // 2), jnp.int32),
            jax.ShapeDtypeStruct((B, D), jnp.float32),
            jax.ShapeDtypeStruct((B, 1), jnp.int32),
            jax.ShapeDtypeStruct((B, 1), jnp.int32),
            jax.ShapeDtypeStruct((B, 1), jnp.float32),
            jax.ShapeDtypeStruct((B, 1), jnp.float32),
        ],
        scratch_shapes=[pltpu.VMEM((1, E), jnp.float32),
                        pltpu.VMEM((BT, BT), jnp.float32)],
    )(x, W_enc, W_gate, b_enc.reshape(1, D), gamma.reshape(1, E))


def _dispatch(sidx0, sidx1, wg0, wg1, enc, B, D, E, CAP):
    TPW = B // _NW
    TCH = 128
    NCH = TPW // TCH
    DP = D // 2

    @functools.partial(
        pl.kernel,
        out_type=(jax.ShapeDtypeStruct(((E + 4) * CAP, DP), jnp.int32),
                  jax.ShapeDtypeStruct(((E + 4) * CAP,), jnp.float32)),
        mesh=_sc_mesh(),
        scratch_types=(
            [pltpu.VMEM((TCH,), jnp.int32)] * 4
            + [pltpu.VMEM((TCH,), jnp.float32)] * 4
            + [pltpu.VMEM((TCH, DP), jnp.int32)] * 2
            + [pltpu.SemaphoreType.DMA] * 4
        ),
    )
    def k(s0_hbm, s1_hbm, w0_hbm, w1_hbm, enc_hbm, buf_hbm, wsl_hbm,
          i0a, i1a, i0b, i1b, w0a, w1a, w0b, w1b, rowsa, rowsb,
          g0, g1, s0, s1):
        wid = lax.axis_index("s") * 2 + lax.axis_index("c")
        bufs = ((i0a, i1a, w0a, w1a, rowsa, g0, s0),
                (i0b, i1b, w0b, w1b, rowsb, g1, s1))

        def start(c, bs):
            i0_v, i1_v, w0_v, w1_v, rows_v, g_sem, _ = bs
            t0 = wid * TPW + c * TCH
            pltpu.async_copy(s0_hbm.at[pl.ds(t0, TCH)], i0_v, g_sem)
            pltpu.async_copy(s1_hbm.at[pl.ds(t0, TCH)], i1_v, g_sem)
            pltpu.async_copy(w0_hbm.at[pl.ds(t0, TCH)], w0_v, g_sem)
            pltpu.async_copy(w1_hbm.at[pl.ds(t0, TCH)], w1_v, g_sem)
            pltpu.async_copy(enc_hbm.at[pl.ds(t0, TCH)], rows_v, g_sem)

        start(0, bufs[0])
        start(1, bufs[1])
        for c in range(NCH):
            i0_v, i1_v, w0_v, w1_v, rows_v, g_sem, s_sem = bufs[c]
            pltpu.make_async_copy(s0_hbm.at[pl.ds(0, TCH)], i0_v, g_sem).wait()
            pltpu.make_async_copy(s1_hbm.at[pl.ds(0, TCH)], i1_v, g_sem).wait()
            pltpu.make_async_copy(w0_hbm.at[pl.ds(0, TCH)], w0_v, g_sem).wait()
            pltpu.make_async_copy(w1_hbm.at[pl.ds(0, TCH)], w1_v, g_sem).wait()
            pltpu.make_async_copy(enc_hbm.at[pl.ds(0, TCH)], rows_v,
                                  g_sem).wait()
            pltpu.async_copy(rows_v, buf_hbm.at[i0_v], s_sem)
            pltpu.async_copy(rows_v, buf_hbm.at[i1_v], s_sem)
            pltpu.async_copy(w0_v, wsl_hbm.at[i0_v], s_sem)
            pltpu.async_copy(w1_v, wsl_hbm.at[i1_v], s_sem)
        for c in range(NCH):
            i0_v, i1_v, w0_v, w1_v, rows_v, g_sem, s_sem = bufs[c]
            pltpu.make_async_copy(rows_v, buf_hbm.at[pl.ds(0, TCH)],
                                  s_sem).wait()
            pltpu.make_async_copy(rows_v, buf_hbm.at[pl.ds(0, TCH)],
                                  s_sem).wait()
            pltpu.make_async_copy(w0_v, wsl_hbm.at[pl.ds(0, TCH)],
                                  s_sem).wait()
            pltpu.make_async_copy(w1_v, wsl_hbm.at[pl.ds(0, TCH)],
                                  s_sem).wait()

    return k(sidx0, sidx1, wg0, wg1, enc)


def _experts(buf, U, V, wslot, E, CAP, D, R):
    EG = 4
    NG = E // EG

    def body(buf_ref, u_ref, v_ref, w_ref, out_ref):
        e = pl.program_id(0)
        xb = pltpu.bitcast(buf_ref[...].reshape(EG * CAP, 1, D // 2),
                           jnp.bfloat16).reshape(EG * CAP, D)
        outs = []
        for g in range(EG):
            xg = xb[g * CAP:(g + 1) * CAP]
            z = jnp.dot(xg, u_ref[g], preferred_element_type=jnp.float32)
            h = z * (1.0 / (1.0 + jnp.exp(-z)))
            outs.append(jnp.dot(h.astype(jnp.bfloat16), v_ref[g],
                                preferred_element_type=jnp.float32))
        dlt = jnp.concatenate(outs, axis=0) * w_ref[0]
        out_ref[...] = jnp.where(e == NG, 0.0, dlt)

    return pl.pallas_call(
        body,
        grid=(NG + 1,),
        in_specs=[
            pl.BlockSpec((EG * CAP, D // 2), lambda e: (e, 0)),
            pl.BlockSpec((EG, D, R), lambda e: (jnp.minimum(e, NG - 1), 0, 0)),
            pl.BlockSpec((EG, R, D), lambda e: (jnp.minimum(e, NG - 1), 0, 0)),
            pl.BlockSpec((1, EG * CAP, 1), lambda e: (e, 0, 0)),
        ],
        out_specs=pl.BlockSpec((EG * CAP, D), lambda e: (e, 0)),
        out_shape=jax.ShapeDtypeStruct(((E + EG) * CAP, D), jnp.float32),
    )(buf, U.astype(jnp.bfloat16), V.astype(jnp.bfloat16),
      wslot.reshape(E // EG + 1, EG * CAP, 1))


def _combine(sidx0, sidx1, y0, dscaled, B, D):
    TCH = 16
    TPW = B // _NW
    NCH = TPW // TCH
    NV = D // _LANES

    @functools.partial(
        pl.kernel,
        out_type=jax.ShapeDtypeStruct((B, D), jnp.float32),
        mesh=_sc_mesh(),
        scratch_types=[
            pltpu.VMEM((TCH,), jnp.int32),
            pltpu.VMEM((TCH,), jnp.int32),
            pltpu.VMEM((TCH,), jnp.int32),
            pltpu.VMEM((TCH,), jnp.int32),
            pltpu.VMEM((TCH, D), jnp.float32),
            pltpu.VMEM((TCH, D), jnp.float32),
            pltpu.VMEM((TCH, D), jnp.float32),
            pltpu.VMEM((TCH, D), jnp.float32),
            pltpu.VMEM((TCH, D), jnp.float32),
            pltpu.VMEM((TCH, D), jnp.float32),
            pltpu.VMEM((TCH, D), jnp.float32),
            pltpu.VMEM((TCH, D), jnp.float32),
            pltpu.SemaphoreType.DMA,
            pltpu.SemaphoreType.DMA,
            pltpu.SemaphoreType.DMA,
            pltpu.SemaphoreType.DMA,
        ],
    )
    def k(s0_hbm, s1_hbm, y0_hbm, dsc_hbm, y_hbm,
          i0a, i1a, i0b, i1b, d0a, d1a, y0a, yoa, d0b, d1b, y0b, yob,
          ga, gb, wa, wb):
        wid = lax.axis_index("s") * 2 + lax.axis_index("c")
        t0 = wid * TPW
        bufs = ((i0a, i1a, d0a, d1a, y0a, yoa, ga, wa),
                (i0b, i1b, d0b, d1b, y0b, yob, gb, wb))

        def start(c, bs):
            i0_v, i1_v, d0_v, d1_v, y0_v, _, g_sem, _ = bs
            tb = t0 + c * TCH
            pltpu.sync_copy(s0_hbm.at[pl.ds(tb, TCH)], i0_v)
            pltpu.sync_copy(s1_hbm.at[pl.ds(tb, TCH)], i1_v)
            pltpu.async_copy(dsc_hbm.at[i0_v], d0_v, g_sem)
            pltpu.async_copy(dsc_hbm.at[i1_v], d1_v, g_sem)
            pltpu.async_copy(y0_hbm.at[pl.ds(tb, TCH)], y0_v, g_sem)

        for b in range(2):
            start(b, bufs[b])

        def phase(j, c, bs):
            _, _, d0_v, d1_v, y0_v, yo_v, g_sem, w_sem = bs
            tb = t0 + c * TCH
            pltpu.make_async_copy(dsc_hbm.at[pl.ds(0, TCH)], d0_v, g_sem).wait()
            pltpu.make_async_copy(dsc_hbm.at[pl.ds(0, TCH)], d1_v, g_sem).wait()
            pltpu.make_async_copy(y0_hbm.at[pl.ds(0, TCH)], y0_v, g_sem).wait()

            @pl.when(j > 0)
            def _():
                pltpu.make_async_copy(yo_v, y_hbm.at[pl.ds(0, TCH)],
                                      w_sem).wait()

            def vloop(q, _):
                t = q // NV
                o = (q % NV) * _LANES
                yo_v[t, pl.ds(o, _LANES)] = (y0_v[t, pl.ds(o, _LANES)]
                                             + d0_v[t, pl.ds(o, _LANES)]
                                             + d1_v[t, pl.ds(o, _LANES)])
                return 0

            lax.fori_loop(0, TCH * NV, vloop, 0)
            pltpu.async_copy(yo_v, y_hbm.at[pl.ds(tb, TCH)], w_sem)

            @pl.when(c + 2 < NCH)
            def _():
                start(c + 2, bs)

        def body(j, _):
            phase(j, 2 * j, bufs[0])
            phase(j, 2 * j + 1, bufs[1])
            return 0

        lax.fori_loop(0, NCH // 2, body, 0)
        for b in range(2):
            _, _, _, _, _, yo_v, _, w_sem = bufs[b]
            pltpu.make_async_copy(yo_v, y_hbm.at[pl.ds(0, TCH)], w_sem).wait()

    return k(sidx0, sidx1, y0, dscaled)


def kernel(x, W_enc, b_enc, W_gate, U, V, gamma):
    B, D = x.shape
    E = W_gate.shape[0]
    R = U.shape[2]
    CAP = int(1.25 * B * _K / E)
    BT = 512

    enc_bf, y0, s1, s2, w1, w2 = _stage_a(x, W_enc, b_enc, W_gate, gamma,
                                          BT, E, CAP)
    sidx0 = s1.reshape(B)
    sidx1 = s2.reshape(B)
    wg0 = w1.reshape(B)
    wg1 = w2.reshape(B)
    buf, wslot = _dispatch(sidx0, sidx1, wg0, wg1, enc_bf, B, D, E, CAP)
    dscaled = _experts(buf, U, V, wslot, E, CAP, D, R)
    y = _combine(sidx0, sidx1, y0, dscaled, B, D)
    return y
